# baseline (speedup 1.0000x reference)
import functools

import jax
import jax.numpy as jnp
from jax import lax
from jax.experimental import pallas as pl
from jax.experimental.pallas import tpu as pltpu
from jax.experimental.pallas import tpu_sc as plsc

N = 10000
E = 320000
D = 128
EPS = 1e-5

N_PAD = 10240
DSLICE = N_PAD // 16
ASLICE = N_PAD // 16

SUB = 125
NSUB1 = 8
CPW1 = 10

CH = 125
CPW3 = 80
PHASES = 2
PHCH = CPW3 // PHASES
BLKCH = 10
NBLK = PHCH // BLKCH
ALAST = N - 15 * ASLICE

_mesh = plsc.VectorSubcoreMesh(core_axis_name="c", subcore_axis_name="s")


@functools.partial(
    pl.kernel,
    out_type=(jax.ShapeDtypeStruct((N_PAD,), jnp.float32),
              jax.ShapeDtypeStruct((N_PAD,), jnp.float32)),
    mesh=_mesh,
    scratch_types=[
        pltpu.VMEM_SHARED((N_PAD,), jnp.float32),
        pltpu.VMEM((CPW1, NSUB1, SUB), jnp.int32),
        pltpu.VMEM((NSUB1, SUB), jnp.float32),
        pltpu.SemaphoreType.DMA,
    ],
)
def _deg(dst_hbm, ones_u_hbm, ones_n_hbm, zeros_n_hbm, d0_hbm, d1_hbm,
         deg_sh, idx_v, ones_v, ssem):
    c = lax.axis_index("c")
    s = lax.axis_index("s")

    @pl.when(c == 0)
    def _():
        pltpu.sync_copy(ones_n_hbm.at[pl.ds(s * DSLICE, DSLICE)],
                        deg_sh.at[pl.ds(s * DSLICE, DSLICE)])

    @pl.when(c == 1)
    def _():
        pltpu.sync_copy(zeros_n_hbm.at[pl.ds(s * DSLICE, DSLICE)],
                        deg_sh.at[pl.ds(s * DSLICE, DSLICE)])

    pltpu.sync_copy(ones_u_hbm, ones_v)
    pltpu.sync_copy(dst_hbm.at[c, s], idx_v)
    plsc.subcore_barrier()

    def body(k, carry):
        descs = []
        for r in range(NSUB1):
            descs.append(pltpu.async_copy(
                ones_v.at[0], deg_sh.at[idx_v.at[k, r]], ssem, add=True))
        for d in descs:
            d.wait()
        return carry

    lax.fori_loop(0, CPW1, body, 0)
    plsc.subcore_barrier()

    @pl.when(c == 0)
    def _():
        pltpu.sync_copy(deg_sh.at[pl.ds(s * DSLICE, DSLICE)],
                        d0_hbm.at[pl.ds(s * DSLICE, DSLICE)])

    @pl.when(c == 1)
    def _():
        pltpu.sync_copy(deg_sh.at[pl.ds(s * DSLICE, DSLICE)],
                        d1_hbm.at[pl.ds(s * DSLICE, DSLICE)])


def _u_body(x_ref, w_ref, h_ref, w1_ref, u_ref, t1_ref):
    u_ref[...] = jnp.dot(x_ref[...], w_ref[...],
                         preferred_element_type=jnp.float32)
    t1_ref[...] = jnp.dot(h_ref[...], w1_ref[...],
                          preferred_element_type=jnp.float32)


def _u_kernel(x, W, H, w1):
    nb = 5
    br = N // nb
    return pl.pallas_call(
        _u_body,
        grid=(nb,),
        in_specs=[
            pl.BlockSpec((br, D), lambda i: (i, 0)),
            pl.BlockSpec((D, D), lambda i: (0, 0)),
            pl.BlockSpec((br, D), lambda i: (i, 0)),
            pl.BlockSpec((D, 2 * D), lambda i: (0, 0)),
        ],
        out_specs=[
            pl.BlockSpec((br, D), lambda i: (i, 0)),
            pl.BlockSpec((br, 2 * D), lambda i: (i, 0)),
        ],
        out_shape=[
            jax.ShapeDtypeStruct((N, D), jnp.float32),
            jax.ShapeDtypeStruct((N, 2 * D), jnp.float32),
        ],
    )(x, W, H, w1)


def _scale_body(u_ref, d0_ref, d1_ref, g_ref, dinv_ref):
    dv = lax.rsqrt(d0_ref[...] + d1_ref[...])
    dinv_ref[...] = dv
    g_ref[...] = dv * u_ref[...]


def _scale_kernel(u, d0s, d1s):
    nb = 5
    br = N // nb
    return pl.pallas_call(
        _scale_body,
        grid=(nb,),
        in_specs=[
            pl.BlockSpec((br, D), lambda i: (i, 0)),
            pl.BlockSpec((br, 1), lambda i: (i, 0)),
            pl.BlockSpec((br, 1), lambda i: (i, 0)),
        ],
        out_specs=[
            pl.BlockSpec((br, D), lambda i: (i, 0)),
            pl.BlockSpec((br, 1), lambda i: (i, 0)),
        ],
        out_shape=[
            jax.ShapeDtypeStruct((N, D), jnp.float32),
            jax.ShapeDtypeStruct((N, 1), jnp.float32),
        ],
    )(u, d0s, d1s)


@functools.partial(
    pl.kernel,
    out_type=(jax.ShapeDtypeStruct((N, D), jnp.float32),
              jax.ShapeDtypeStruct((N, D), jnp.float32)),
    mesh=_mesh,
    scratch_types=[
        pltpu.VMEM_SHARED((N, D), jnp.float32),
        pltpu.VMEM((PHCH, CH), jnp.int32),
        pltpu.VMEM((PHCH, CH), jnp.int32),
        pltpu.VMEM((2, CH, D), jnp.float32),
        pltpu.SemaphoreType.DMA,
        pltpu.SemaphoreType.DMA,
        pltpu.SemaphoreType.DMA,
        pltpu.SemaphoreType.DMA,
    ],
)
def _agg(g_hbm, src_hbm, dst_hbm, zeros_hbm, p0_hbm, p1_hbm,
         acc_sh, src_v, dst_v, rows_v, gsem0, gsem1, ssem0, ssem1):
    c = lax.axis_index("c")
    s = lax.axis_index("s")
    w = c * 16 + s
    gsem = (gsem0, gsem1)
    ssem = (ssem0, ssem1)

    @pl.when(s < 15)
    def _():
        pltpu.sync_copy(zeros_hbm, acc_sh.at[pl.ds(s * ASLICE, ASLICE)])

    @pl.when(s == 15)
    def _():
        pltpu.sync_copy(zeros_hbm.at[pl.ds(0, ALAST)],
                        acc_sh.at[pl.ds(15 * ASLICE, ALAST)])

    plsc.subcore_barrier()

    def body(m, carry):
        base = m * BLKCH
        gd = [None, None]
        sd = [None, None]
        gd[0] = pltpu.async_copy(g_hbm.at[src_v.at[base]], rows_v.at[0],
                                 gsem[0])
        for t in range(BLKCH):
            b = t % 2
            if t < BLKCH - 1:
                o = 1 - b
                if sd[o] is not None:
                    sd[o].wait()
                gd[o] = pltpu.async_copy(g_hbm.at[src_v.at[base + t + 1]],
                                         rows_v.at[o], gsem[o])
            gd[b].wait()
            sd[b] = pltpu.async_copy(rows_v.at[b],
                                     acc_sh.at[dst_v.at[base + t]],
                                     ssem[b], add=True)
        sd[0].wait()
        sd[1].wait()
        return carry

    for ph in range(PHASES):
        pltpu.sync_copy(src_hbm.at[w, ph], src_v)
        pltpu.sync_copy(dst_hbm.at[w, ph], dst_v)
        lax.fori_loop(0, NBLK, body, 0)

    plsc.subcore_barrier()

    def writeout(dst):
        @pl.when(s < 15)
        def _():
            pltpu.sync_copy(acc_sh.at[pl.ds(s * ASLICE, ASLICE)],
                            dst.at[pl.ds(s * ASLICE, ASLICE)])

        @pl.when(s == 15)
        def _():
            pltpu.sync_copy(acc_sh.at[pl.ds(15 * ASLICE, ALAST)],
                            dst.at[pl.ds(15 * ASLICE, ALAST)])

    @pl.when(c == 0)
    def _():
        writeout(p0_hbm)

    @pl.when(c == 1)
    def _():
        writeout(p1_hbm)


def _final_body(p0_ref, p1_ref, g_ref, dinv_ref, b_ref, gamma_ref, beta_ref,
                a_ref, h_ref, t1_ref, wh1_ref, bzrh_ref, w0_ref,
                out_ref, ssum, ssq):
    k = pl.program_id(0)
    j = pl.program_id(1)

    hc = dinv_ref[...] * (p0_ref[...] + p1_ref[...] + g_ref[...]) + b_ref[...]

    @pl.when(jnp.logical_and(k == 0, j == 0))
    def _():
        ssum[...] = jnp.zeros_like(ssum)
        ssq[...] = jnp.zeros_like(ssq)

    @pl.when(k == 0)
    def _():
        ssum[...] += jnp.sum(hc, axis=0, keepdims=True)
        ssq[...] += jnp.sum(hc * hc, axis=0, keepdims=True)

    @pl.when(k == 1)
    def _():
        mean = ssum[...] * (1.0 / N)
        var = ssq[...] * (1.0 / N) - mean * mean
        hb = (gamma_ref[...] * (hc - mean) * lax.rsqrt(var + EPS)
              + beta_ref[...])
        hp = jnp.where(hb >= 0.0, hb, a_ref[...] * hb)
        Hb = h_ref[...]
        t0 = (jnp.dot(hp, w0_ref[...], preferred_element_type=jnp.float32)
              + bzrh_ref[...])
        t1 = t1_ref[...]
        z = jax.nn.sigmoid(t0[:, 0:D] + t1[:, 0:D])
        r = jax.nn.sigmoid(t0[:, D:2 * D] + t1[:, D:2 * D])
        ht = jnp.tanh(
            t0[:, 2 * D:3 * D]
            + jnp.dot(r * Hb, wh1_ref[...],
                      preferred_element_type=jnp.float32))
        out_ref[...] = z * Hb + (1.0 - z) * ht


def _final_kernel(p0, p1, g, dinv2d, b, gamma, beta, a, H, t1,
                  Wz, Wr, Wh, bz, brr, bh):
    nb = 5
    br = N // nb
    row_spec = pl.BlockSpec((br, D), lambda k, j: (j, 0))
    vec_spec = pl.BlockSpec((1, D), lambda k, j: (0, 0))
    lazy_row_spec = pl.BlockSpec((br, D), lambda k, j: (j * k, 0))
    lazy_t1_spec = pl.BlockSpec((br, 2 * D), lambda k, j: (j * k, 0))
    w0 = jnp.concatenate([Wz[0:D], Wr[0:D], Wh[0:D]], axis=1)
    bzrh = jnp.concatenate([bz, brr, bh]).reshape(1, 3 * D)
    return pl.pallas_call(
        _final_body,
        grid=(2, nb),
        in_specs=[
            row_spec, row_spec, row_spec,
            pl.BlockSpec((br, 1), lambda k, j: (j, 0)),
            vec_spec, vec_spec, vec_spec,
            pl.BlockSpec((1, 1), lambda k, j: (0, 0)),
            lazy_row_spec,
            lazy_t1_spec,
            pl.BlockSpec((D, D), lambda k, j: (0, 0)),
            pl.BlockSpec((1, 3 * D), lambda k, j: (0, 0)),
            pl.BlockSpec((D, 3 * D), lambda k, j: (0, 0)),
        ],
        out_specs=row_spec,
        out_shape=jax.ShapeDtypeStruct((N, D), jnp.float32),
        scratch_shapes=[
            pltpu.VMEM((1, D), jnp.float32),
            pltpu.VMEM((1, D), jnp.float32),
        ],
    )(p0, p1, g, dinv2d, b.reshape(1, D), gamma.reshape(1, D),
      beta.reshape(1, D), a.reshape(1, 1), H, t1, Wh[D:], bzrh, w0)


@jax.jit
def kernel(x, edge_index, H, W, b, gamma, beta, a, Wz, bz, Wr, br, Wh, bh):
    dst1 = edge_index[1].reshape(2, 16, CPW1, NSUB1, SUB)
    src3 = edge_index[0].reshape(32, PHASES, PHCH, CH)
    dst3 = edge_index[1].reshape(32, PHASES, PHCH, CH)
    ones_u = jnp.ones((NSUB1, SUB), jnp.float32)
    ones_n = jnp.ones((N_PAD,), jnp.float32)
    zeros_n = jnp.zeros((N_PAD,), jnp.float32)

    w1 = jnp.concatenate([Wz[D:], Wr[D:]], axis=1)
    u, t1 = _u_kernel(x, W, H, w1)
    d0, d1 = _deg(dst1, ones_u, ones_n, zeros_n)

    g, dinv2d = _scale_kernel(u, d0[:N].reshape(N, 1), d1[:N].reshape(N, 1))

    zeros = jnp.zeros((N_PAD // 16, D), jnp.float32)
    p0, p1 = _agg(g, src3, dst3, zeros)

    return _final_kernel(p0, p1, g, dinv2d, b, gamma, beta, a, H, t1,
                         Wz, Wr, Wh, bz, br, bh)


# speedup vs baseline: 1.0272x; 1.0272x over previous
import functools

import jax
import jax.numpy as jnp
from jax import lax
from jax.experimental import pallas as pl
from jax.experimental.pallas import tpu as pltpu
from jax.experimental.pallas import tpu_sc as plsc

N = 10000
E = 320000
D = 128
EPS = 1e-5

N_PAD = 10240
DSLICE = N_PAD // 16
ASLICE = N_PAD // 16

SUB = 125
NSUB1 = 8
CPW1 = 10

CH = 100
CPW3 = 100
PHASES = 5
PHCH = CPW3 // PHASES
BLKCH = 10
NBLK = PHCH // BLKCH
NROT = 3
ALAST = N - 15 * ASLICE

_mesh = plsc.VectorSubcoreMesh(core_axis_name="c", subcore_axis_name="s")


@functools.partial(
    pl.kernel,
    out_type=(jax.ShapeDtypeStruct((N_PAD,), jnp.float32),
              jax.ShapeDtypeStruct((N_PAD,), jnp.float32)),
    mesh=_mesh,
    scratch_types=[
        pltpu.VMEM_SHARED((N_PAD,), jnp.float32),
        pltpu.VMEM((CPW1, NSUB1, SUB), jnp.int32),
        pltpu.VMEM((NSUB1, SUB), jnp.float32),
        pltpu.SemaphoreType.DMA,
    ],
)
def _deg(dst_hbm, ones_u_hbm, ones_n_hbm, zeros_n_hbm, d0_hbm, d1_hbm,
         deg_sh, idx_v, ones_v, ssem):
    c = lax.axis_index("c")
    s = lax.axis_index("s")

    @pl.when(c == 0)
    def _():
        pltpu.sync_copy(ones_n_hbm.at[pl.ds(s * DSLICE, DSLICE)],
                        deg_sh.at[pl.ds(s * DSLICE, DSLICE)])

    @pl.when(c == 1)
    def _():
        pltpu.sync_copy(zeros_n_hbm.at[pl.ds(s * DSLICE, DSLICE)],
                        deg_sh.at[pl.ds(s * DSLICE, DSLICE)])

    pltpu.sync_copy(ones_u_hbm, ones_v)
    pltpu.sync_copy(dst_hbm.at[c, s], idx_v)
    plsc.subcore_barrier()

    def body(k, carry):
        descs = []
        for r in range(NSUB1):
            descs.append(pltpu.async_copy(
                ones_v.at[0], deg_sh.at[idx_v.at[k, r]], ssem, add=True))
        for d in descs:
            d.wait()
        return carry

    lax.fori_loop(0, CPW1, body, 0)
    plsc.subcore_barrier()

    @pl.when(c == 0)
    def _():
        pltpu.sync_copy(deg_sh.at[pl.ds(s * DSLICE, DSLICE)],
                        d0_hbm.at[pl.ds(s * DSLICE, DSLICE)])

    @pl.when(c == 1)
    def _():
        pltpu.sync_copy(deg_sh.at[pl.ds(s * DSLICE, DSLICE)],
                        d1_hbm.at[pl.ds(s * DSLICE, DSLICE)])


def _g_body(x_ref, w_ref, d0_ref, d1_ref, g_ref, dinv_ref):
    dv = lax.rsqrt(d0_ref[...] + d1_ref[...])
    dinv_ref[...] = dv
    g_ref[...] = dv * jnp.dot(x_ref[...], w_ref[...],
                              preferred_element_type=jnp.float32)


def _g_kernel(x, W, d0s, d1s):
    nb = 10
    br = N // nb
    return pl.pallas_call(
        _g_body,
        grid=(nb,),
        in_specs=[
            pl.BlockSpec((br, D), lambda i: (i, 0)),
            pl.BlockSpec((D, D), lambda i: (0, 0)),
            pl.BlockSpec((br, 1), lambda i: (i, 0)),
            pl.BlockSpec((br, 1), lambda i: (i, 0)),
        ],
        out_specs=[
            pl.BlockSpec((br, D), lambda i: (i, 0)),
            pl.BlockSpec((br, 1), lambda i: (i, 0)),
        ],
        out_shape=[
            jax.ShapeDtypeStruct((N, D), jnp.float32),
            jax.ShapeDtypeStruct((N, 1), jnp.float32),
        ],
    )(x, W, d0s, d1s)


@functools.partial(
    pl.kernel,
    out_type=(jax.ShapeDtypeStruct((N, D), jnp.float32),
              jax.ShapeDtypeStruct((N, D), jnp.float32)),
    mesh=_mesh,
    scratch_types=[
        pltpu.VMEM_SHARED((N, D), jnp.float32),
        pltpu.VMEM((PHCH, CH), jnp.int32),
        pltpu.VMEM((PHCH, CH), jnp.int32),
        pltpu.VMEM((NROT, CH, D), jnp.float32),
        pltpu.SemaphoreType.DMA,
        pltpu.SemaphoreType.DMA,
        pltpu.SemaphoreType.DMA,
        pltpu.SemaphoreType.DMA,
        pltpu.SemaphoreType.DMA,
        pltpu.SemaphoreType.DMA,
    ],
)
def _agg(g_hbm, src_hbm, dst_hbm, zeros_hbm, p0_hbm, p1_hbm,
         acc_sh, src_v, dst_v, rows_v, gsem0, gsem1, gsem2,
         ssem0, ssem1, ssem2):
    c = lax.axis_index("c")
    s = lax.axis_index("s")
    w = c * 16 + s
    gsem = (gsem0, gsem1, gsem2)
    ssem = (ssem0, ssem1, ssem2)

    @pl.when(s < 15)
    def _():
        pltpu.sync_copy(zeros_hbm.at[pl.ds(s * ASLICE, ASLICE)],
                        acc_sh.at[pl.ds(s * ASLICE, ASLICE)])

    @pl.when(s == 15)
    def _():
        pltpu.sync_copy(zeros_hbm.at[pl.ds(15 * ASLICE, ALAST)],
                        acc_sh.at[pl.ds(15 * ASLICE, ALAST)])

    plsc.subcore_barrier()

    def body(m, carry):
        base = m * BLKCH
        gd = [None] * NROT
        sd = [None] * NROT
        gd[0] = pltpu.async_copy(g_hbm.at[src_v.at[base]], rows_v.at[0],
                                 gsem[0])
        gd[1] = pltpu.async_copy(g_hbm.at[src_v.at[base + 1]],
                                 rows_v.at[1], gsem[1])
        for t in range(BLKCH):
            b = t % NROT
            if t < BLKCH - 2:
                o = (t + 2) % NROT
                if sd[o] is not None:
                    sd[o].wait()
                gd[o] = pltpu.async_copy(g_hbm.at[src_v.at[base + t + 2]],
                                         rows_v.at[o], gsem[o])
            gd[b].wait()
            sd[b] = pltpu.async_copy(rows_v.at[b],
                                     acc_sh.at[dst_v.at[base + t]],
                                     ssem[b], add=True)
        for q in range(NROT):
            if sd[q] is not None:
                sd[q].wait()
        return carry

    for ph in range(PHASES):
        pltpu.sync_copy(src_hbm.at[w, ph], src_v)
        pltpu.sync_copy(dst_hbm.at[w, ph], dst_v)
        lax.fori_loop(0, NBLK, body, 0)

    plsc.subcore_barrier()

    def writeout(dst):
        @pl.when(s < 15)
        def _():
            pltpu.sync_copy(acc_sh.at[pl.ds(s * ASLICE, ASLICE)],
                            dst.at[pl.ds(s * ASLICE, ASLICE)])

        @pl.when(s == 15)
        def _():
            pltpu.sync_copy(acc_sh.at[pl.ds(15 * ASLICE, ALAST)],
                            dst.at[pl.ds(15 * ASLICE, ALAST)])

    @pl.when(c == 0)
    def _():
        writeout(p0_hbm)

    @pl.when(c == 1)
    def _():
        writeout(p1_hbm)


def _final_body(p0_ref, p1_ref, g_ref, dinv_ref, b_ref, gamma_ref, beta_ref,
                a_ref, h_ref, w0_ref, w1_ref, wh1_ref, bzrh_ref,
                out_ref, ssum, ssq):
    k = pl.program_id(0)
    j = pl.program_id(1)

    hc = dinv_ref[...] * (p0_ref[...] + p1_ref[...] + g_ref[...]) + b_ref[...]

    @pl.when(jnp.logical_and(k == 0, j == 0))
    def _():
        ssum[...] = jnp.zeros_like(ssum)
        ssq[...] = jnp.zeros_like(ssq)

    @pl.when(k == 0)
    def _():
        ssum[...] += jnp.sum(hc, axis=0, keepdims=True)
        ssq[...] += jnp.sum(hc * hc, axis=0, keepdims=True)

    @pl.when(k == 1)
    def _():
        mean = ssum[...] * (1.0 / N)
        var = ssq[...] * (1.0 / N) - mean * mean
        hb = (gamma_ref[...] * (hc - mean) * lax.rsqrt(var + EPS)
              + beta_ref[...])
        hp = jnp.where(hb >= 0.0, hb, a_ref[...] * hb)
        Hb = h_ref[...]
        t0 = (jnp.dot(hp, w0_ref[...], preferred_element_type=jnp.float32)
              + bzrh_ref[...])
        t1 = jnp.dot(Hb, w1_ref[...], preferred_element_type=jnp.float32)
        z = jax.nn.sigmoid(t0[:, 0:D] + t1[:, 0:D])
        r = jax.nn.sigmoid(t0[:, D:2 * D] + t1[:, D:2 * D])
        ht = jnp.tanh(
            t0[:, 2 * D:3 * D]
            + jnp.dot(r * Hb, wh1_ref[...],
                      preferred_element_type=jnp.float32))
        out_ref[...] = z * Hb + (1.0 - z) * ht


def _final_kernel(p0, p1, g, dinv2d, b, gamma, beta, a, H,
                  Wz, Wr, Wh, bz, brr, bh):
    nb = 5
    br = N // nb
    row_spec = pl.BlockSpec((br, D), lambda k, j: (j, 0))
    vec_spec = pl.BlockSpec((1, D), lambda k, j: (0, 0))
    lazy_row_spec = pl.BlockSpec((br, D), lambda k, j: (j * k, 0))
    w0 = jnp.concatenate([Wz[0:D], Wr[0:D], Wh[0:D]], axis=1)
    w1 = jnp.concatenate([Wz[D:], Wr[D:]], axis=1)
    bzrh = jnp.concatenate([bz, brr, bh]).reshape(1, 3 * D)
    return pl.pallas_call(
        _final_body,
        grid=(2, nb),
        in_specs=[
            row_spec, row_spec, row_spec,
            pl.BlockSpec((br, 1), lambda k, j: (j, 0)),
            vec_spec, vec_spec, vec_spec,
            pl.BlockSpec((1, 1), lambda k, j: (0, 0)),
            lazy_row_spec,
            pl.BlockSpec((D, 3 * D), lambda k, j: (0, 0)),
            pl.BlockSpec((D, 2 * D), lambda k, j: (0, 0)),
            pl.BlockSpec((D, D), lambda k, j: (0, 0)),
            pl.BlockSpec((1, 3 * D), lambda k, j: (0, 0)),
        ],
        out_specs=row_spec,
        out_shape=jax.ShapeDtypeStruct((N, D), jnp.float32),
        scratch_shapes=[
            pltpu.VMEM((1, D), jnp.float32),
            pltpu.VMEM((1, D), jnp.float32),
        ],
    )(p0, p1, g, dinv2d, b.reshape(1, D), gamma.reshape(1, D),
      beta.reshape(1, D), a.reshape(1, 1), H, w0, w1, Wh[D:], bzrh)


@jax.jit
def kernel(x, edge_index, H, W, b, gamma, beta, a, Wz, bz, Wr, br, Wh, bh):
    dst1 = edge_index[1].reshape(2, 16, CPW1, NSUB1, SUB)
    src3 = edge_index[0].reshape(32, PHASES, PHCH, CH)
    dst3 = edge_index[1].reshape(32, PHASES, PHCH, CH)
    ones_u = jnp.ones((NSUB1, SUB), jnp.float32)
    ones_n = jnp.ones((N_PAD,), jnp.float32)
    zeros_n = jnp.zeros((N_PAD,), jnp.float32)
    d0, d1 = _deg(dst1, ones_u, ones_n, zeros_n)

    g, dinv2d = _g_kernel(x, W, d0[:N].reshape(N, 1), d1[:N].reshape(N, 1))

    zeros = jnp.zeros((N, D), jnp.float32)
    p0, p1 = _agg(g, src3, dst3, zeros)

    return _final_kernel(p0, p1, g, dinv2d, b, gamma, beta, a, H,
                         Wz, Wr, Wh, bz, br, bh)


# speedup vs baseline: 1.0402x; 1.0126x over previous
import functools

import jax
import jax.numpy as jnp
from jax import lax
from jax.experimental import pallas as pl
from jax.experimental.pallas import tpu as pltpu
from jax.experimental.pallas import tpu_sc as plsc

N = 10000
E = 320000
D = 128
EPS = 1e-5

N_PAD = 10240
DSLICE = N_PAD // 16
ASLICE = N_PAD // 16

SUB = 125
NSUB1 = 8
CPW1 = 10

CH = 125
CPW3 = 80
PHASES = 2
PHCH = CPW3 // PHASES
BLKCH = 10
NBLK = PHCH // BLKCH
ALAST = N - 15 * ASLICE

_mesh = plsc.VectorSubcoreMesh(core_axis_name="c", subcore_axis_name="s")


@functools.partial(
    pl.kernel,
    out_type=(jax.ShapeDtypeStruct((N_PAD,), jnp.float32),
              jax.ShapeDtypeStruct((N_PAD,), jnp.float32)),
    mesh=_mesh,
    scratch_types=[
        pltpu.VMEM_SHARED((N_PAD,), jnp.float32),
        pltpu.VMEM((CPW1, NSUB1, SUB), jnp.int32),
        pltpu.VMEM((NSUB1, SUB), jnp.float32),
        pltpu.SemaphoreType.DMA,
    ],
)
def _deg(dst_hbm, ones_u_hbm, ones_n_hbm, zeros_n_hbm, d0_hbm, d1_hbm,
         deg_sh, idx_v, ones_v, ssem):
    c = lax.axis_index("c")
    s = lax.axis_index("s")

    @pl.when(c == 0)
    def _():
        pltpu.sync_copy(ones_n_hbm.at[pl.ds(s * DSLICE, DSLICE)],
                        deg_sh.at[pl.ds(s * DSLICE, DSLICE)])

    @pl.when(c == 1)
    def _():
        pltpu.sync_copy(zeros_n_hbm.at[pl.ds(s * DSLICE, DSLICE)],
                        deg_sh.at[pl.ds(s * DSLICE, DSLICE)])

    pltpu.sync_copy(ones_u_hbm, ones_v)
    pltpu.sync_copy(dst_hbm.at[c, s], idx_v)
    plsc.subcore_barrier()

    def body(k, carry):
        descs = []
        for r in range(NSUB1):
            descs.append(pltpu.async_copy(
                ones_v.at[0], deg_sh.at[idx_v.at[k, r]], ssem, add=True))
        for d in descs:
            d.wait()
        return carry

    lax.fori_loop(0, CPW1, body, 0)
    plsc.subcore_barrier()

    @pl.when(c == 0)
    def _():
        pltpu.sync_copy(deg_sh.at[pl.ds(s * DSLICE, DSLICE)],
                        d0_hbm.at[pl.ds(s * DSLICE, DSLICE)])

    @pl.when(c == 1)
    def _():
        pltpu.sync_copy(deg_sh.at[pl.ds(s * DSLICE, DSLICE)],
                        d1_hbm.at[pl.ds(s * DSLICE, DSLICE)])


def _g_body(x_ref, w_ref, d0_ref, d1_ref, g_ref, dinv_ref):
    dv = lax.rsqrt(d0_ref[...] + d1_ref[...])
    dinv_ref[...] = dv
    g_ref[...] = dv * jnp.dot(x_ref[...], w_ref[...],
                              preferred_element_type=jnp.float32)


def _g_kernel(x, W, d0s, d1s):
    nb = 10
    br = N // nb
    return pl.pallas_call(
        _g_body,
        grid=(nb,),
        in_specs=[
            pl.BlockSpec((br, D), lambda i: (i, 0)),
            pl.BlockSpec((D, D), lambda i: (0, 0)),
            pl.BlockSpec((br, 1), lambda i: (i, 0)),
            pl.BlockSpec((br, 1), lambda i: (i, 0)),
        ],
        out_specs=[
            pl.BlockSpec((br, D), lambda i: (i, 0)),
            pl.BlockSpec((br, 1), lambda i: (i, 0)),
        ],
        out_shape=[
            jax.ShapeDtypeStruct((N, D), jnp.float32),
            jax.ShapeDtypeStruct((N, 1), jnp.float32),
        ],
    )(x, W, d0s, d1s)


@functools.partial(
    pl.kernel,
    out_type=(jax.ShapeDtypeStruct((N, D), jnp.float32),
              jax.ShapeDtypeStruct((N, D), jnp.float32)),
    mesh=_mesh,
    scratch_types=[
        pltpu.VMEM_SHARED((N, D), jnp.float32),
        pltpu.VMEM((PHCH, CH), jnp.int32),
        pltpu.VMEM((PHCH, CH), jnp.int32),
        pltpu.VMEM((2, CH, D), jnp.float32),
        pltpu.SemaphoreType.DMA,
        pltpu.SemaphoreType.DMA,
        pltpu.SemaphoreType.DMA,
        pltpu.SemaphoreType.DMA,
    ],
)
def _agg(g_hbm, src_hbm, dst_hbm, zeros_hbm, p0_hbm, p1_hbm,
         acc_sh, src_v, dst_v, rows_v, gsem0, gsem1, ssem0, ssem1):
    c = lax.axis_index("c")
    s = lax.axis_index("s")
    w = c * 16 + s
    gsem = (gsem0, gsem1)
    ssem = (ssem0, ssem1)

    @pl.when(s < 15)
    def _():
        pltpu.sync_copy(zeros_hbm.at[pl.ds(s * ASLICE, ASLICE)],
                        acc_sh.at[pl.ds(s * ASLICE, ASLICE)])

    @pl.when(s == 15)
    def _():
        pltpu.sync_copy(zeros_hbm.at[pl.ds(15 * ASLICE, ALAST)],
                        acc_sh.at[pl.ds(15 * ASLICE, ALAST)])

    plsc.subcore_barrier()

    def body(m, carry):
        base = m * BLKCH
        gd = [None, None]
        sd = [None, None]
        gd[0] = pltpu.async_copy(g_hbm.at[src_v.at[base]], rows_v.at[0],
                                 gsem[0])
        for t in range(BLKCH):
            b = t % 2
            if t < BLKCH - 1:
                o = 1 - b
                if sd[o] is not None:
                    sd[o].wait()
                gd[o] = pltpu.async_copy(g_hbm.at[src_v.at[base + t + 1]],
                                         rows_v.at[o], gsem[o])
            gd[b].wait()
            sd[b] = pltpu.async_copy(rows_v.at[b],
                                     acc_sh.at[dst_v.at[base + t]],
                                     ssem[b], add=True)
        sd[0].wait()
        sd[1].wait()
        return carry

    for ph in range(PHASES):
        pltpu.sync_copy(src_hbm.at[w, ph], src_v)
        pltpu.sync_copy(dst_hbm.at[w, ph], dst_v)
        lax.fori_loop(0, NBLK, body, 0)

    plsc.subcore_barrier()

    def writeout(dst):
        @pl.when(s < 15)
        def _():
            pltpu.sync_copy(acc_sh.at[pl.ds(s * ASLICE, ASLICE)],
                            dst.at[pl.ds(s * ASLICE, ASLICE)])

        @pl.when(s == 15)
        def _():
            pltpu.sync_copy(acc_sh.at[pl.ds(15 * ASLICE, ALAST)],
                            dst.at[pl.ds(15 * ASLICE, ALAST)])

    @pl.when(c == 0)
    def _():
        writeout(p0_hbm)

    @pl.when(c == 1)
    def _():
        writeout(p1_hbm)


def _final_body(p0_ref, p1_ref, g_ref, dinv_ref, b_ref, gamma_ref, beta_ref,
                a_ref, h_ref, w0_ref, w1_ref, wh1_ref, bzrh_ref,
                out_ref, ssum, ssq):
    k = pl.program_id(0)
    j = pl.program_id(1)

    hc = dinv_ref[...] * (p0_ref[...] + p1_ref[...] + g_ref[...]) + b_ref[...]

    @pl.when(jnp.logical_and(k == 0, j == 0))
    def _():
        ssum[...] = jnp.zeros_like(ssum)
        ssq[...] = jnp.zeros_like(ssq)

    @pl.when(k == 0)
    def _():
        ssum[...] += jnp.sum(hc, axis=0, keepdims=True)
        ssq[...] += jnp.sum(hc * hc, axis=0, keepdims=True)

    @pl.when(k == 1)
    def _():
        mean = ssum[...] * (1.0 / N)
        var = ssq[...] * (1.0 / N) - mean * mean
        hb = (gamma_ref[...] * (hc - mean) * lax.rsqrt(var + EPS)
              + beta_ref[...])
        hp = jnp.where(hb >= 0.0, hb, a_ref[...] * hb)
        Hb = h_ref[...]
        t0 = (jnp.dot(hp, w0_ref[...], preferred_element_type=jnp.float32)
              + bzrh_ref[...])
        t1 = jnp.dot(Hb, w1_ref[...], preferred_element_type=jnp.float32)
        z = jax.nn.sigmoid(t0[:, 0:D] + t1[:, 0:D])
        r = jax.nn.sigmoid(t0[:, D:2 * D] + t1[:, D:2 * D])
        ht = jnp.tanh(
            t0[:, 2 * D:3 * D]
            + jnp.dot(r * Hb, wh1_ref[...],
                      preferred_element_type=jnp.float32))
        out_ref[...] = z * Hb + (1.0 - z) * ht


def _final_kernel(p0, p1, g, dinv2d, b, gamma, beta, a, H,
                  Wz, Wr, Wh, bz, brr, bh):
    nb = 5
    br = N // nb
    row_spec = pl.BlockSpec((br, D), lambda k, j: (j, 0))
    vec_spec = pl.BlockSpec((1, D), lambda k, j: (0, 0))
    lazy_row_spec = pl.BlockSpec((br, D), lambda k, j: (j * k, 0))
    w0 = jnp.concatenate([Wz[0:D], Wr[0:D], Wh[0:D]], axis=1)
    w1 = jnp.concatenate([Wz[D:], Wr[D:]], axis=1)
    bzrh = jnp.concatenate([bz, brr, bh]).reshape(1, 3 * D)
    return pl.pallas_call(
        _final_body,
        grid=(2, nb),
        in_specs=[
            row_spec, row_spec, row_spec,
            pl.BlockSpec((br, 1), lambda k, j: (j, 0)),
            vec_spec, vec_spec, vec_spec,
            pl.BlockSpec((1, 1), lambda k, j: (0, 0)),
            lazy_row_spec,
            pl.BlockSpec((D, 3 * D), lambda k, j: (0, 0)),
            pl.BlockSpec((D, 2 * D), lambda k, j: (0, 0)),
            pl.BlockSpec((D, D), lambda k, j: (0, 0)),
            pl.BlockSpec((1, 3 * D), lambda k, j: (0, 0)),
        ],
        out_specs=row_spec,
        out_shape=jax.ShapeDtypeStruct((N, D), jnp.float32),
        scratch_shapes=[
            pltpu.VMEM((1, D), jnp.float32),
            pltpu.VMEM((1, D), jnp.float32),
        ],
    )(p0, p1, g, dinv2d, b.reshape(1, D), gamma.reshape(1, D),
      beta.reshape(1, D), a.reshape(1, 1), H, w0, w1, Wh[D:], bzrh)


@jax.jit
def kernel(x, edge_index, H, W, b, gamma, beta, a, Wz, bz, Wr, br, Wh, bh):
    dst1 = edge_index[1].reshape(2, 16, CPW1, NSUB1, SUB)
    src3 = edge_index[0].reshape(32, PHASES, PHCH, CH)
    dst3 = edge_index[1].reshape(32, PHASES, PHCH, CH)
    ones_u = jnp.ones((NSUB1, SUB), jnp.float32)
    ones_n = jnp.ones((N_PAD,), jnp.float32)
    zeros_n = jnp.zeros((N_PAD,), jnp.float32)
    d0, d1 = _deg(dst1, ones_u, ones_n, zeros_n)

    g, dinv2d = _g_kernel(x, W, d0[:N].reshape(N, 1), d1[:N].reshape(N, 1))

    zeros = jnp.zeros((N, D), jnp.float32)
    p0, p1 = _agg(g, src3, dst3, zeros)

    return _final_kernel(p0, p1, g, dinv2d, b, gamma, beta, a, H,
                         Wz, Wr, Wh, bz, br, bh)


# speedup vs baseline: 1.0459x; 1.0055x over previous
import functools

import jax
import jax.numpy as jnp
from jax import lax
from jax.experimental import pallas as pl
from jax.experimental.pallas import tpu as pltpu
from jax.experimental.pallas import tpu_sc as plsc

N = 10000
E = 320000
D = 128
EPS = 1e-5

N_PAD = 10240
DSLICE = N_PAD // 16
ASLICE = N_PAD // 16

SUB = 125
NSUB1 = 8
CPW1 = 10

CH = 125
CPW3 = 80
PHASES = 2
PHCH = CPW3 // PHASES
BLKCH = 10
NBLK = PHCH // BLKCH
ALAST = N - 15 * ASLICE

_mesh = plsc.VectorSubcoreMesh(core_axis_name="c", subcore_axis_name="s")


@functools.partial(
    pl.kernel,
    out_type=(jax.ShapeDtypeStruct((N_PAD,), jnp.float32),
              jax.ShapeDtypeStruct((N_PAD,), jnp.float32)),
    mesh=_mesh,
    scratch_types=[
        pltpu.VMEM_SHARED((N_PAD,), jnp.float32),
        pltpu.VMEM((CPW1, NSUB1, SUB), jnp.int32),
        pltpu.VMEM((NSUB1, SUB), jnp.float32),
        pltpu.SemaphoreType.DMA,
    ],
)
def _deg(dst_hbm, ones_u_hbm, ones_n_hbm, zeros_n_hbm, d0_hbm, d1_hbm,
         deg_sh, idx_v, ones_v, ssem):
    c = lax.axis_index("c")
    s = lax.axis_index("s")

    @pl.when(c == 0)
    def _():
        pltpu.sync_copy(ones_n_hbm.at[pl.ds(s * DSLICE, DSLICE)],
                        deg_sh.at[pl.ds(s * DSLICE, DSLICE)])

    @pl.when(c == 1)
    def _():
        pltpu.sync_copy(zeros_n_hbm.at[pl.ds(s * DSLICE, DSLICE)],
                        deg_sh.at[pl.ds(s * DSLICE, DSLICE)])

    pltpu.sync_copy(ones_u_hbm, ones_v)
    pltpu.sync_copy(dst_hbm.at[c, s], idx_v)
    plsc.subcore_barrier()

    def body(k, carry):
        descs = []
        for kk in range(2):
            for r in range(NSUB1):
                descs.append(pltpu.async_copy(
                    ones_v.at[0], deg_sh.at[idx_v.at[2 * k + kk, r]],
                    ssem, add=True))
        for d in descs:
            d.wait()
        return carry

    lax.fori_loop(0, CPW1 // 2, body, 0)
    plsc.subcore_barrier()

    @pl.when(c == 0)
    def _():
        pltpu.sync_copy(deg_sh.at[pl.ds(s * DSLICE, DSLICE)],
                        d0_hbm.at[pl.ds(s * DSLICE, DSLICE)])

    @pl.when(c == 1)
    def _():
        pltpu.sync_copy(deg_sh.at[pl.ds(s * DSLICE, DSLICE)],
                        d1_hbm.at[pl.ds(s * DSLICE, DSLICE)])


def _g_body(x_ref, w_ref, d0_ref, d1_ref, g_ref, dinv_ref):
    dv = lax.rsqrt(d0_ref[...] + d1_ref[...])
    dinv_ref[...] = dv
    g_ref[...] = dv * jnp.dot(x_ref[...], w_ref[...],
                              preferred_element_type=jnp.float32)


def _g_kernel(x, W, d0s, d1s):
    nb = 5
    br = N // nb
    return pl.pallas_call(
        _g_body,
        grid=(nb,),
        in_specs=[
            pl.BlockSpec((br, D), lambda i: (i, 0)),
            pl.BlockSpec((D, D), lambda i: (0, 0)),
            pl.BlockSpec((br, 1), lambda i: (i, 0)),
            pl.BlockSpec((br, 1), lambda i: (i, 0)),
        ],
        out_specs=[
            pl.BlockSpec((br, D), lambda i: (i, 0)),
            pl.BlockSpec((br, 1), lambda i: (i, 0)),
        ],
        out_shape=[
            jax.ShapeDtypeStruct((N, D), jnp.float32),
            jax.ShapeDtypeStruct((N, 1), jnp.float32),
        ],
    )(x, W, d0s, d1s)


@functools.partial(
    pl.kernel,
    out_type=(jax.ShapeDtypeStruct((N, D), jnp.float32),
              jax.ShapeDtypeStruct((N, D), jnp.float32)),
    mesh=_mesh,
    scratch_types=[
        pltpu.VMEM_SHARED((N, D), jnp.float32),
        pltpu.VMEM((PHCH, CH), jnp.int32),
        pltpu.VMEM((PHCH, CH), jnp.int32),
        pltpu.VMEM((2, CH, D), jnp.float32),
        pltpu.SemaphoreType.DMA,
        pltpu.SemaphoreType.DMA,
        pltpu.SemaphoreType.DMA,
        pltpu.SemaphoreType.DMA,
    ],
)
def _agg(g_hbm, src_hbm, dst_hbm, zeros_hbm, p0_hbm, p1_hbm,
         acc_sh, src_v, dst_v, rows_v, gsem0, gsem1, ssem0, ssem1):
    c = lax.axis_index("c")
    s = lax.axis_index("s")
    w = c * 16 + s
    gsem = (gsem0, gsem1)
    ssem = (ssem0, ssem1)

    @pl.when(s < 15)
    def _():
        pltpu.sync_copy(zeros_hbm.at[pl.ds(s * ASLICE, ASLICE)],
                        acc_sh.at[pl.ds(s * ASLICE, ASLICE)])

    @pl.when(s == 15)
    def _():
        pltpu.sync_copy(zeros_hbm.at[pl.ds(15 * ASLICE, ALAST)],
                        acc_sh.at[pl.ds(15 * ASLICE, ALAST)])

    plsc.subcore_barrier()

    def body(m, carry):
        base = m * BLKCH
        gd = [None, None]
        sd = [None, None]
        gd[0] = pltpu.async_copy(g_hbm.at[src_v.at[base]], rows_v.at[0],
                                 gsem[0])
        for t in range(BLKCH):
            b = t % 2
            if t < BLKCH - 1:
                o = 1 - b
                if sd[o] is not None:
                    sd[o].wait()
                gd[o] = pltpu.async_copy(g_hbm.at[src_v.at[base + t + 1]],
                                         rows_v.at[o], gsem[o])
            gd[b].wait()
            sd[b] = pltpu.async_copy(rows_v.at[b],
                                     acc_sh.at[dst_v.at[base + t]],
                                     ssem[b], add=True)
        sd[0].wait()
        sd[1].wait()
        return carry

    for ph in range(PHASES):
        pltpu.sync_copy(src_hbm.at[w, ph], src_v)
        pltpu.sync_copy(dst_hbm.at[w, ph], dst_v)
        lax.fori_loop(0, NBLK, body, 0)

    plsc.subcore_barrier()

    def writeout(dst):
        @pl.when(s < 15)
        def _():
            pltpu.sync_copy(acc_sh.at[pl.ds(s * ASLICE, ASLICE)],
                            dst.at[pl.ds(s * ASLICE, ASLICE)])

        @pl.when(s == 15)
        def _():
            pltpu.sync_copy(acc_sh.at[pl.ds(15 * ASLICE, ALAST)],
                            dst.at[pl.ds(15 * ASLICE, ALAST)])

    @pl.when(c == 0)
    def _():
        writeout(p0_hbm)

    @pl.when(c == 1)
    def _():
        writeout(p1_hbm)


def _final_body(p0_ref, p1_ref, g_ref, dinv_ref, b_ref, gamma_ref, beta_ref,
                a_ref, h_ref, w0_ref, w1_ref, wh1_ref, bzrh_ref,
                out_ref, ssum, ssq):
    k = pl.program_id(0)
    j = pl.program_id(1)

    hc = dinv_ref[...] * (p0_ref[...] + p1_ref[...] + g_ref[...]) + b_ref[...]

    @pl.when(jnp.logical_and(k == 0, j == 0))
    def _():
        ssum[...] = jnp.zeros_like(ssum)
        ssq[...] = jnp.zeros_like(ssq)

    @pl.when(k == 0)
    def _():
        ssum[...] += jnp.sum(hc, axis=0, keepdims=True)
        ssq[...] += jnp.sum(hc * hc, axis=0, keepdims=True)

    @pl.when(k == 1)
    def _():
        mean = ssum[...] * (1.0 / N)
        var = ssq[...] * (1.0 / N) - mean * mean
        hb = (gamma_ref[...] * (hc - mean) * lax.rsqrt(var + EPS)
              + beta_ref[...])
        hp = jnp.where(hb >= 0.0, hb, a_ref[...] * hb)
        Hb = h_ref[...]
        t0 = (jnp.dot(hp, w0_ref[...], preferred_element_type=jnp.float32)
              + bzrh_ref[...])
        t1 = jnp.dot(Hb, w1_ref[...], preferred_element_type=jnp.float32)
        z = jax.nn.sigmoid(t0[:, 0:D] + t1[:, 0:D])
        r = jax.nn.sigmoid(t0[:, D:2 * D] + t1[:, D:2 * D])
        ht = jnp.tanh(
            t0[:, 2 * D:3 * D]
            + jnp.dot(r * Hb, wh1_ref[...],
                      preferred_element_type=jnp.float32))
        out_ref[...] = z * Hb + (1.0 - z) * ht


def _final_kernel(p0, p1, g, dinv2d, b, gamma, beta, a, H,
                  Wz, Wr, Wh, bz, brr, bh):
    nb = 5
    br = N // nb
    row_spec = pl.BlockSpec((br, D), lambda k, j: (j, 0))
    vec_spec = pl.BlockSpec((1, D), lambda k, j: (0, 0))
    lazy_row_spec = pl.BlockSpec((br, D), lambda k, j: (j * k, 0))
    w0 = jnp.concatenate([Wz[0:D], Wr[0:D], Wh[0:D]], axis=1)
    w1 = jnp.concatenate([Wz[D:], Wr[D:]], axis=1)
    bzrh = jnp.concatenate([bz, brr, bh]).reshape(1, 3 * D)
    return pl.pallas_call(
        _final_body,
        grid=(2, nb),
        in_specs=[
            row_spec, row_spec, row_spec,
            pl.BlockSpec((br, 1), lambda k, j: (j, 0)),
            vec_spec, vec_spec, vec_spec,
            pl.BlockSpec((1, 1), lambda k, j: (0, 0)),
            lazy_row_spec,
            pl.BlockSpec((D, 3 * D), lambda k, j: (0, 0)),
            pl.BlockSpec((D, 2 * D), lambda k, j: (0, 0)),
            pl.BlockSpec((D, D), lambda k, j: (0, 0)),
            pl.BlockSpec((1, 3 * D), lambda k, j: (0, 0)),
        ],
        out_specs=row_spec,
        out_shape=jax.ShapeDtypeStruct((N, D), jnp.float32),
        scratch_shapes=[
            pltpu.VMEM((1, D), jnp.float32),
            pltpu.VMEM((1, D), jnp.float32),
        ],
    )(p0, p1, g, dinv2d, b.reshape(1, D), gamma.reshape(1, D),
      beta.reshape(1, D), a.reshape(1, 1), H, w0, w1, Wh[D:], bzrh)


@jax.jit
def kernel(x, edge_index, H, W, b, gamma, beta, a, Wz, bz, Wr, br, Wh, bh):
    dst1 = edge_index[1].reshape(2, 16, CPW1, NSUB1, SUB)
    src3 = edge_index[0].reshape(32, PHASES, PHCH, CH)
    dst3 = edge_index[1].reshape(32, PHASES, PHCH, CH)
    ones_u = jnp.ones((NSUB1, SUB), jnp.float32)
    ones_n = jnp.ones((N_PAD,), jnp.float32)
    zeros_n = jnp.zeros((N_PAD,), jnp.float32)
    d0, d1 = _deg(dst1, ones_u, ones_n, zeros_n)

    g, dinv2d = _g_kernel(x, W, d0[:N].reshape(N, 1), d1[:N].reshape(N, 1))

    zeros = jnp.zeros((N, D), jnp.float32)
    p0, p1 = _agg(g, src3, dst3, zeros)

    return _final_kernel(p0, p1, g, dinv2d, b, gamma, beta, a, H,
                         Wz, Wr, Wh, bz, br, bh)
